# baseline (speedup 1.0000x reference)
import functools

import jax
import jax.numpy as jnp
import numpy as np
from jax import lax
from jax.experimental import pallas as pl
from jax.experimental.pallas import tpu as pltpu
from jax.experimental.pallas import tpu_sc as plsc

_T = 8192
_D = 2048
_DROP_RATIO = 0.1
_NEW_LEN = int(_T * (1.0 - _DROP_RATIO))

_NC = 2
_NS = 16
_NW = _NC * _NS
_CHUNK = 8
_NCHUNK = 29
_B_PER_W = _CHUNK * _NCHUNK
_NBUF = 6
_AHEAD = 3


def _tf2x32(k1, k2, x0, x1):
    rot0 = (13, 15, 26, 6)
    rot1 = (17, 29, 16, 24)
    k1 = np.uint32(k1)
    k2 = np.uint32(k2)
    ks = (k1, k2, np.uint32(k1 ^ k2 ^ np.uint32(0x1BD11BDA)))
    x0 = x0.astype(np.uint32) + ks[0]
    x1 = x1.astype(np.uint32) + ks[1]

    def rnd(x0, x1, r):
        x0 = x0 + x1
        x1 = (x1 << np.uint32(r)) | (x1 >> np.uint32(32 - r))
        return x0, x0 ^ x1

    for i, rots in enumerate((rot0, rot1, rot0, rot1, rot0)):
        for r in rots:
            x0, x1 = rnd(x0, x1, r)
        x0 = x0 + ks[(i + 1) % 3]
        x1 = x1 + ks[(i + 2) % 3] + np.uint32(i + 1)
    return x0, x1


def _split2(k1, k2):
    c1 = np.zeros(2, np.uint32)
    c2 = np.arange(2, dtype=np.uint32)
    b1, b2 = _tf2x32(k1, k2, c1, c2)
    return [(b1[i], b2[i]) for i in range(2)]


def _random_bits_32(k1, k2, n):
    c1 = np.zeros(n, np.uint32)
    c2 = np.arange(n, dtype=np.uint32)
    b1, b2 = _tf2x32(k1, k2, c1, c2)
    return b1 ^ b2


def _permutation_from_seed(seed, n):
    k1 = np.uint32(np.uint64(seed) >> np.uint64(32))
    k2 = np.uint32(seed & 0xFFFFFFFF)
    _, key = _split2(k1, k2)
    x = np.arange(n, dtype=np.int32)
    num_rounds = int(np.ceil(3 * np.log(n) / np.log(2**32 - 1)))
    for _ in range(num_rounds):
        key, subkey = _split2(*key)
        sort_keys = _random_bits_32(subkey[0], subkey[1], n)
        x = x[np.argsort(sort_keys, kind="stable")]
    return x


@functools.lru_cache(maxsize=None)
def _index_tables() -> np.ndarray:
    perm = _permutation_from_seed(42, _T)
    idx = np.sort(perm[:_NEW_LEN]).astype(np.int32)
    bases = np.minimum(np.arange(_NW) * _B_PER_W, _NEW_LEN - _B_PER_W)
    out_rows = (bases[:, None] + np.arange(_B_PER_W)[None, :]).astype(np.int32)
    in_rows = idx[out_rows]
    both = np.stack([in_rows, out_rows], axis=1)
    return both.reshape(_NW, 2 * _NCHUNK, _CHUNK)


def _pipeline(gather, scatter):
    g = {c: gather(c) for c in range(_AHEAD)}
    w = {}
    waited = set()
    for c in range(_NCHUNK):
        g[c].wait()
        w[c] = scatter(c)
        n = c + _AHEAD
        if n < _NCHUNK:
            prev = n - _NBUF
            if prev >= 0:
                w[prev].wait()
                waited.add(prev)
            g[n] = gather(n)
    for c in range(_NCHUNK):
        if c not in waited:
            w[c].wait()


def _gather_body(x_hbm, idx_hbm, out_hbm, idx_v, bufs, gsems, wsems):
    wid = lax.axis_index("s") * _NC + lax.axis_index("c")
    pltpu.sync_copy(idx_hbm.at[wid], idx_v)

    def gather(c):
        return pltpu.async_copy(
            x_hbm.at[idx_v.at[c]], bufs[c % _NBUF], gsems[c % _NBUF])

    def scatter_indirect(c):
        return pltpu.async_copy(
            bufs[c % _NBUF], out_hbm.at[idx_v.at[_NCHUNK + c]],
            wsems[c % _NBUF])

    base = pl.multiple_of(wid * _B_PER_W, _CHUNK)

    def scatter_linear(c):
        return pltpu.async_copy(
            bufs[c % _NBUF], out_hbm.at[pl.ds(base + c * _CHUNK, _CHUNK)],
            wsems[c % _NBUF])

    @pl.when(wid < _NW - 1)
    def _():
        _pipeline(gather, scatter_linear)

    @pl.when(wid == _NW - 1)
    def _():
        _pipeline(gather, scatter_indirect)


@functools.partial(
    pl.kernel,
    out_type=jax.ShapeDtypeStruct((_NEW_LEN, _D), jnp.float32),
    mesh=plsc.VectorSubcoreMesh(core_axis_name="c", subcore_axis_name="s"),
    scratch_types=(
        [pltpu.VMEM((2 * _NCHUNK, _CHUNK), jnp.int32)]
        + [pltpu.VMEM((_CHUNK, _D), jnp.float32)] * _NBUF
        + [pltpu.SemaphoreType.DMA] * (2 * _NBUF)
    ),
)
def _sc_gather(x_hbm, idx_hbm, out_hbm, idx_v, *rest):
    bufs = rest[:_NBUF]
    gsems = rest[_NBUF:2 * _NBUF]
    wsems = rest[2 * _NBUF:]
    _gather_body(x_hbm, idx_hbm, out_hbm, idx_v, bufs, gsems, wsems)


def kernel(x):
    return _sc_gather(x, jnp.asarray(_index_tables()))


# speedup vs baseline: 1.0346x; 1.0346x over previous
import functools

import jax
import jax.numpy as jnp
import numpy as np
from jax import lax
from jax.experimental import pallas as pl
from jax.experimental.pallas import tpu as pltpu
from jax.experimental.pallas import tpu_sc as plsc

_T = 8192
_D = 2048
_DROP_RATIO = 0.1
_NEW_LEN = int(_T * (1.0 - _DROP_RATIO))

_NC = 2
_NS = 16
_NW = _NC * _NS
_CHUNK = 8
_NCHUNK = 29
_B_PER_W = _CHUNK * _NCHUNK
_NBUF = 7
_AHEAD = 4


def _tf2x32(k1, k2, x0, x1):
    rot0 = (13, 15, 26, 6)
    rot1 = (17, 29, 16, 24)
    k1 = np.uint32(k1)
    k2 = np.uint32(k2)
    ks = (k1, k2, np.uint32(k1 ^ k2 ^ np.uint32(0x1BD11BDA)))
    x0 = x0.astype(np.uint32) + ks[0]
    x1 = x1.astype(np.uint32) + ks[1]

    def rnd(x0, x1, r):
        x0 = x0 + x1
        x1 = (x1 << np.uint32(r)) | (x1 >> np.uint32(32 - r))
        return x0, x0 ^ x1

    for i, rots in enumerate((rot0, rot1, rot0, rot1, rot0)):
        for r in rots:
            x0, x1 = rnd(x0, x1, r)
        x0 = x0 + ks[(i + 1) % 3]
        x1 = x1 + ks[(i + 2) % 3] + np.uint32(i + 1)
    return x0, x1


def _split2(k1, k2):
    c1 = np.zeros(2, np.uint32)
    c2 = np.arange(2, dtype=np.uint32)
    b1, b2 = _tf2x32(k1, k2, c1, c2)
    return [(b1[i], b2[i]) for i in range(2)]


def _random_bits_32(k1, k2, n):
    c1 = np.zeros(n, np.uint32)
    c2 = np.arange(n, dtype=np.uint32)
    b1, b2 = _tf2x32(k1, k2, c1, c2)
    return b1 ^ b2


def _permutation_from_seed(seed, n):
    k1 = np.uint32(np.uint64(seed) >> np.uint64(32))
    k2 = np.uint32(seed & 0xFFFFFFFF)
    _, key = _split2(k1, k2)
    x = np.arange(n, dtype=np.int32)
    num_rounds = int(np.ceil(3 * np.log(n) / np.log(2**32 - 1)))
    for _ in range(num_rounds):
        key, subkey = _split2(*key)
        sort_keys = _random_bits_32(subkey[0], subkey[1], n)
        x = x[np.argsort(sort_keys, kind="stable")]
    return x


@functools.lru_cache(maxsize=None)
def _index_tables() -> np.ndarray:
    perm = _permutation_from_seed(42, _T)
    idx = np.sort(perm[:_NEW_LEN]).astype(np.int32)
    bases = np.minimum(np.arange(_NW) * _B_PER_W, _NEW_LEN - _B_PER_W)
    out_rows = (bases[:, None] + np.arange(_B_PER_W)[None, :]).astype(np.int32)
    in_rows = idx[out_rows]
    both = np.stack([in_rows, out_rows], axis=1)
    return both.reshape(_NW, 2 * _NCHUNK, _CHUNK)


def _pipeline(gather, scatter):
    g = {c: gather(c) for c in range(_AHEAD)}
    w = {}
    waited = set()
    for c in range(_NCHUNK):
        g[c].wait()
        w[c] = scatter(c)
        n = c + _AHEAD
        if n < _NCHUNK:
            prev = n - _NBUF
            if prev >= 0:
                w[prev].wait()
                waited.add(prev)
            g[n] = gather(n)
    for c in range(_NCHUNK):
        if c not in waited:
            w[c].wait()


def _gather_body(x_hbm, idx_hbm, out_hbm, idx_v, bufs, gsems, wsems):
    wid = lax.axis_index("s") * _NC + lax.axis_index("c")
    pltpu.sync_copy(idx_hbm.at[wid], idx_v)

    def gather(c):
        return pltpu.async_copy(
            x_hbm.at[idx_v.at[c]], bufs[c % _NBUF], gsems[c % _NBUF])

    def scatter(c):
        return pltpu.async_copy(
            bufs[c % _NBUF], out_hbm.at[idx_v.at[_NCHUNK + c]],
            wsems[c % _NBUF])

    _pipeline(gather, scatter)


@functools.partial(
    pl.kernel,
    out_type=jax.ShapeDtypeStruct((_NEW_LEN, _D), jnp.float32),
    mesh=plsc.VectorSubcoreMesh(core_axis_name="c", subcore_axis_name="s"),
    scratch_types=(
        [pltpu.VMEM((2 * _NCHUNK, _CHUNK), jnp.int32)]
        + [pltpu.VMEM((_CHUNK, _D), jnp.float32)] * _NBUF
        + [pltpu.SemaphoreType.DMA] * (2 * _NBUF)
    ),
)
def _sc_gather(x_hbm, idx_hbm, out_hbm, idx_v, *rest):
    bufs = rest[:_NBUF]
    gsems = rest[_NBUF:2 * _NBUF]
    wsems = rest[2 * _NBUF:]
    _gather_body(x_hbm, idx_hbm, out_hbm, idx_v, bufs, gsems, wsems)


def kernel(x):
    return _sc_gather(x, jnp.asarray(_index_tables()))
